# Initial kernel scaffold; baseline (speedup 1.0000x reference)
import jax
import jax.numpy as jnp
from jax.experimental import pallas as pl


def kernel(x, edge_index, edge_attr, batch, proj_w, proj_b, vn_emb, conv0_w, conv0_b, bn0_g, bn0_b, conv1_w, conv1_b, bn1_g, bn1_b, conv2_w, conv2_b, bn2_g, bn2_b, mlp0_w1, mlp0_b1, mlp0_bn1_g, mlp0_bn1_b, mlp0_w2, mlp0_b2, mlp0_bn2_g, mlp0_bn2_b, mlp1_w1, mlp1_b1, mlp1_bn1_g, mlp1_bn1_b, mlp1_w2, mlp1_b2, mlp1_bn2_g, mlp1_bn2_b):
    raise NotImplementedError("write your pallas kernel here")


# speedup vs baseline: 9.1714x; 9.1714x over previous
import functools

import jax
import jax.numpy as jnp
from jax import lax
from jax.experimental import pallas as pl
from jax.experimental.pallas import tpu as pltpu
from jax.experimental.pallas import tpu_sc as plsc

N = 10000
E = 320000
H = 128
NG = 64
NC = 2
NS = 16
NW = NC * NS
ER = E // 128
RB = ER // NW
RR = ER - RB * NW

_MESH = plsc.VectorSubcoreMesh(core_axis_name="c", subcore_axis_name="s",
                               num_cores=NC, num_subcores=NS)


def _worker(c, s):
    w = s * NC + c
    start = RB * w + jnp.minimum(w, RR)
    nrows = RB + (w < RR).astype(jnp.int32)
    return start, nrows


def _sc_deg(row2d, ones128, zeros_n):
    @functools.partial(
        pl.kernel,
        out_type=jax.ShapeDtypeStruct((NC, N), jnp.float32),
        mesh=_MESH,
        scratch_types=[pltpu.VMEM((128,), jnp.int32),
                       pltpu.VMEM((128,), jnp.float32),
                       pltpu.VMEM_SHARED((N,), jnp.float32)],
    )
    def k(row_h, ones_h, zeros_h, out_h, idx_v, ones_v, acc_sh):
        c = lax.axis_index("c")
        s = lax.axis_index("s")
        @pl.when(s == 0)
        def _():
            pltpu.sync_copy(zeros_h, acc_sh)
        pltpu.sync_copy(ones_h, ones_v)
        plsc.subcore_barrier()
        start, nrows = _worker(c, s)

        def body(i, carry):
            @pl.when(i < nrows)
            def _():
                pltpu.sync_copy(row_h.at[start + i], idx_v)
                pltpu.sync_copy(ones_v, acc_sh.at[idx_v], add=True)
            return carry

        lax.fori_loop(0, RB + 1, body, 0)
        plsc.subcore_barrier()
        @pl.when(s == 0)
        def _():
            pltpu.sync_copy(acc_sh, out_h.at[c])

    return k(row2d, ones128, zeros_n)


def _sc_dsrow(row2d, dis):
    @functools.partial(
        pl.kernel,
        out_type=jax.ShapeDtypeStruct((E,), jnp.float32),
        mesh=_MESH,
        scratch_types=[pltpu.VMEM((128,), jnp.int32),
                       pltpu.VMEM((128,), jnp.float32)],
    )
    def k(row_h, dis_h, out_h, idx_v, ds_v):
        c = lax.axis_index("c")
        s = lax.axis_index("s")
        start, nrows = _worker(c, s)

        def body(i, carry):
            @pl.when(i < nrows)
            def _():
                r = start + i
                pltpu.sync_copy(row_h.at[r], idx_v)
                pltpu.sync_copy(dis_h.at[idx_v], ds_v)
                pltpu.sync_copy(ds_v, out_h.at[pl.ds(r * 128, 128)])
            return carry

        lax.fori_loop(0, RB + 1, body, 0)

    return k(row2d, dis)


def _sc_scatter_ea(ea3, col2d, zeros2d):
    @functools.partial(
        pl.kernel,
        out_type=jax.ShapeDtypeStruct((NC, N, H), jnp.float32),
        mesh=_MESH,
        scratch_types=[pltpu.VMEM((128,), jnp.int32),
                       pltpu.VMEM((128, H), jnp.float32),
                       pltpu.VMEM_SHARED((N, H), jnp.float32)],
    )
    def k(ea_h, col_h, z2_h, out_h, idx_v, rows_v, acc_sh):
        c = lax.axis_index("c")
        s = lax.axis_index("s")
        @pl.when(s == 0)
        def _():
            pltpu.sync_copy(z2_h, acc_sh)
        plsc.subcore_barrier()
        start, nrows = _worker(c, s)

        def body(i, carry):
            @pl.when(i < nrows)
            def _():
                r = start + i
                pltpu.sync_copy(col_h.at[r], idx_v)
                pltpu.sync_copy(ea_h.at[r], rows_v)
                pltpu.sync_copy(rows_v, acc_sh.at[idx_v], add=True)
            return carry

        lax.fori_loop(0, RB + 1, body, 0)
        plsc.subcore_barrier()
        @pl.when(s == 0)
        def _():
            pltpu.sync_copy(acc_sh, out_h.at[c])

    return k(ea3, col2d, zeros2d)


def _sc_spmm(hs, row2d, col2d, zeros2d):
    @functools.partial(
        pl.kernel,
        out_type=jax.ShapeDtypeStruct((NC, N, H), jnp.float32),
        mesh=_MESH,
        scratch_types=[pltpu.VMEM((128,), jnp.int32),
                       pltpu.VMEM((128,), jnp.int32),
                       pltpu.VMEM((128, H), jnp.float32),
                       pltpu.VMEM_SHARED((N, H), jnp.float32)],
    )
    def k(hs_h, row_h, col_h, z2_h, out_h, idxr_v, idxc_v, rows_v, acc_sh):
        c = lax.axis_index("c")
        s = lax.axis_index("s")
        @pl.when(s == 0)
        def _():
            pltpu.sync_copy(z2_h, acc_sh)
        plsc.subcore_barrier()
        start, nrows = _worker(c, s)

        def body(i, carry):
            @pl.when(i < nrows)
            def _():
                r = start + i
                pltpu.sync_copy(row_h.at[r], idxr_v)
                pltpu.sync_copy(col_h.at[r], idxc_v)
                pltpu.sync_copy(hs_h.at[idxr_v], rows_v)
                pltpu.sync_copy(rows_v, acc_sh.at[idxc_v], add=True)
            return carry

        lax.fori_loop(0, RB + 1, body, 0)
        plsc.subcore_barrier()
        @pl.when(s == 0)
        def _():
            pltpu.sync_copy(acc_sh, out_h.at[c])

    return k(hs, row2d, col2d, zeros2d)


_BLK = 400
_G = N // _BLK


def _tc_ea_proj(ea, pw, pb2, ds):
    blk = 2000

    def body(ea_ref, pw_ref, pb_ref, ds_ref, o_ref):
        p = jnp.dot(ea_ref[...], pw_ref[...],
                    preferred_element_type=jnp.float32) + pb_ref[...]
        o_ref[...] = p * ds_ref[...]

    return pl.pallas_call(
        body, grid=(E // blk,),
        in_specs=[pl.BlockSpec((blk, H), lambda i: (i, 0)),
                  pl.BlockSpec((H, H), lambda i: (0, 0)),
                  pl.BlockSpec((1, H), lambda i: (0, 0)),
                  pl.BlockSpec((blk, 1), lambda i: (i, 0))],
        out_specs=pl.BlockSpec((blk, H), lambda i: (i, 0)),
        out_shape=jax.ShapeDtypeStruct((E, H), jnp.float32),
    )(ea, pw, pb2, ds.reshape(E, 1))


def _tc_pre(src, batch2, vn, dis2, proj=None, bn=None):
    def body(src_ref, b_ref, vn_ref, dis_ref, e1_ref, e2_ref, e3_ref,
             hs_ref, pool_ref, acc):
        i = pl.program_id(0)
        if proj is not None:
            hbase = jnp.dot(src_ref[...], e1_ref[...],
                            preferred_element_type=jnp.float32) + e2_ref[...]
        else:
            st = e1_ref[...]
            mu = st[0:1] * (1.0 / N)
            var = st[1:2] * (1.0 / N) - mu * mu
            hbase = (src_ref[...] - mu) * lax.rsqrt(var + 1e-5) * e2_ref[...] + e3_ref[...]
            hbase = jnp.maximum(hbase, 0.0)
        ids = lax.broadcasted_iota(jnp.int32, (_BLK, NG), 1)
        oh = (b_ref[...] == ids).astype(jnp.float32)
        hl = hbase + jnp.dot(oh, vn_ref[...], preferred_element_type=jnp.float32,
                             precision=lax.Precision.HIGHEST)
        hs_ref[...] = dis_ref[...] * hl
        p = lax.dot_general(oh, hl, (((0,), (0,)), ((), ())),
                            preferred_element_type=jnp.float32,
                            precision=lax.Precision.HIGHEST)
        @pl.when(i == 0)
        def _():
            acc[...] = jnp.zeros_like(acc)
        acc[...] += p
        pool_ref[...] = acc[...]

    if proj is not None:
        e_specs = [pl.BlockSpec((H, H), lambda i: (0, 0)),
                   pl.BlockSpec((1, H), lambda i: (0, 0)),
                   pl.BlockSpec((1, H), lambda i: (0, 0))]
        extra = [proj[0], proj[1], proj[1]]
    else:
        e_specs = [pl.BlockSpec((2, H), lambda i: (0, 0)),
                   pl.BlockSpec((1, H), lambda i: (0, 0)),
                   pl.BlockSpec((1, H), lambda i: (0, 0))]
        extra = [bn[0], bn[1], bn[2]]

    return pl.pallas_call(
        body, grid=(_G,),
        in_specs=[pl.BlockSpec((_BLK, H), lambda i: (i, 0)),
                  pl.BlockSpec((_BLK, 1), lambda i: (i, 0)),
                  pl.BlockSpec((NG, H), lambda i: (0, 0)),
                  pl.BlockSpec((_BLK, 1), lambda i: (i, 0))] + e_specs,
        out_specs=[pl.BlockSpec((_BLK, H), lambda i: (i, 0)),
                   pl.BlockSpec((NG, H), lambda i: (0, 0))],
        out_shape=[jax.ShapeDtypeStruct((N, H), jnp.float32),
                   jax.ShapeDtypeStruct((NG, H), jnp.float32)],
        scratch_shapes=[pltpu.VMEM((NG, H), jnp.float32)],
    )(src, batch2, vn, dis2, *extra)


def _tc_post(raw, c2, W, hs, dis2, cb2):
    def body(raw_ref, c_ref, W_ref, hs_ref, dis_ref, cb_ref,
             pre_ref, st_ref, acc):
        i = pl.program_id(0)
        r = raw_ref[0] + raw_ref[1] + c_ref[0] + c_ref[1]
        aggr = dis_ref[...] * r + dis_ref[...] * hs_ref[...]
        pre = jnp.dot(aggr, W_ref[...],
                      preferred_element_type=jnp.float32) + cb_ref[...]
        pre_ref[...] = pre
        @pl.when(i == 0)
        def _():
            acc[...] = jnp.zeros_like(acc)
        acc[0:1] += pre.sum(0, keepdims=True)
        acc[1:2] += (pre * pre).sum(0, keepdims=True)
        st_ref[...] = acc[...]

    return pl.pallas_call(
        body, grid=(_G,),
        in_specs=[pl.BlockSpec((NC, _BLK, H), lambda i: (0, i, 0)),
                  pl.BlockSpec((NC, _BLK, H), lambda i: (0, i, 0)),
                  pl.BlockSpec((H, H), lambda i: (0, 0)),
                  pl.BlockSpec((_BLK, H), lambda i: (i, 0)),
                  pl.BlockSpec((_BLK, 1), lambda i: (i, 0)),
                  pl.BlockSpec((1, H), lambda i: (0, 0))],
        out_specs=[pl.BlockSpec((_BLK, H), lambda i: (i, 0)),
                   pl.BlockSpec((2, H), lambda i: (0, 0))],
        out_shape=[jax.ShapeDtypeStruct((N, H), jnp.float32),
                   jax.ShapeDtypeStruct((2, H), jnp.float32)],
        scratch_shapes=[pltpu.VMEM((2, H), jnp.float32)],
    )(raw, c2, W, hs, dis2, cb2)


def _tc_vnmlp(pooled, vn, w1, b1, g1, bb1, w2, b2, g2, bb2, gmask2):
    def body(p_ref, vn_ref, w1_ref, b1_ref, g1_ref, bb1_ref,
             w2_ref, b2_ref, g2_ref, bb2_ref, gm_ref, o_ref):
        gm = gm_ref[...]
        cnt = gm.sum()
        t = p_ref[...] + vn_ref[...]
        t = jnp.dot(t, w1_ref[...], preferred_element_type=jnp.float32) + b1_ref[...]
        mu = (t * gm).sum(0, keepdims=True) / cnt
        var = (((t - mu) ** 2) * gm).sum(0, keepdims=True) / cnt
        t = jnp.maximum((t - mu) * lax.rsqrt(var + 1e-5) * g1_ref[...] + bb1_ref[...], 0.0)
        t = jnp.dot(t, w2_ref[...], preferred_element_type=jnp.float32) + b2_ref[...]
        mu2 = (t * gm).sum(0, keepdims=True) / cnt
        var2 = (((t - mu2) ** 2) * gm).sum(0, keepdims=True) / cnt
        o_ref[...] = jnp.maximum((t - mu2) * lax.rsqrt(var2 + 1e-5) * g2_ref[...] + bb2_ref[...], 0.0)

    return pl.pallas_call(
        body, out_shape=jax.ShapeDtypeStruct((NG, H), jnp.float32),
    )(pooled, vn, w1, b1.reshape(1, -1), g1.reshape(1, -1), bb1.reshape(1, -1),
      w2, b2.reshape(1, -1), g2.reshape(1, -1), bb2.reshape(1, -1), gmask2)


def _tc_final(pre, stats, g2, b2):
    def body(pre_ref, st_ref, g_ref, b_ref, o_ref):
        st = st_ref[...]
        mu = st[0:1] * (1.0 / N)
        var = st[1:2] * (1.0 / N) - mu * mu
        o_ref[...] = (pre_ref[...] - mu) * lax.rsqrt(var + 1e-5) * g_ref[...] + b_ref[...]

    return pl.pallas_call(
        body, grid=(_G,),
        in_specs=[pl.BlockSpec((_BLK, H), lambda i: (i, 0)),
                  pl.BlockSpec((2, H), lambda i: (0, 0)),
                  pl.BlockSpec((1, H), lambda i: (0, 0)),
                  pl.BlockSpec((1, H), lambda i: (0, 0))],
        out_specs=pl.BlockSpec((_BLK, H), lambda i: (i, 0)),
        out_shape=jax.ShapeDtypeStruct((N, H), jnp.float32),
    )(pre, stats, g2, b2)


def kernel(x, edge_index, edge_attr, batch,
           proj_w, proj_b, vn_emb,
           conv0_w, conv0_b, bn0_g, bn0_b,
           conv1_w, conv1_b, bn1_g, bn1_b,
           conv2_w, conv2_b, bn2_g, bn2_b,
           mlp0_w1, mlp0_b1, mlp0_bn1_g, mlp0_bn1_b, mlp0_w2, mlp0_b2, mlp0_bn2_g, mlp0_bn2_b,
           mlp1_w1, mlp1_b1, mlp1_bn1_g, mlp1_bn1_b, mlp1_w2, mlp1_b2, mlp1_bn2_g, mlp1_bn2_b):
    row2d = edge_index[0].astype(jnp.int32).reshape(ER, 128)
    col2d = edge_index[1].astype(jnp.int32).reshape(ER, 128)
    zeros_n = jnp.zeros((N,), jnp.float32)
    zeros2d = jnp.zeros((N, H), jnp.float32)
    ones128 = jnp.ones((128,), jnp.float32)

    deg2 = _sc_deg(row2d, ones128, zeros_n)
    dis = (deg2[0] + deg2[1] + 1.0) ** -0.5
    dis2 = dis.reshape(N, 1)
    dsrow = _sc_dsrow(row2d, dis)
    pb2 = proj_b.reshape(1, H)
    ea_s = _tc_ea_proj(edge_attr, proj_w, pb2, dsrow)
    c2 = _sc_scatter_ea(ea_s.reshape(ER, 128, H), col2d, zeros2d)

    batch2 = batch.astype(jnp.int32).reshape(N, 1)
    gmask2 = (jnp.arange(NG) < batch[-1] + 1).astype(jnp.float32).reshape(NG, 1)
    vn = jnp.broadcast_to(vn_emb, (NG, H))

    convs = [(conv0_w, conv0_b, bn0_g, bn0_b),
             (conv1_w, conv1_b, bn1_g, bn1_b),
             (conv2_w, conv2_b, bn2_g, bn2_b)]
    mlps = [(mlp0_w1, mlp0_b1, mlp0_bn1_g, mlp0_bn1_b, mlp0_w2, mlp0_b2, mlp0_bn2_g, mlp0_bn2_b),
            (mlp1_w1, mlp1_b1, mlp1_bn1_g, mlp1_bn1_b, mlp1_w2, mlp1_b2, mlp1_bn2_g, mlp1_bn2_b)]

    pre = None
    stats = None
    for l in range(3):
        W, cb, _, _ = convs[l]
        if l == 0:
            hs, pooled = _tc_pre(x, batch2, vn, dis2, proj=(proj_w, pb2))
        else:
            pg, pb_ = convs[l - 1][2], convs[l - 1][3]
            hs, pooled = _tc_pre(pre, batch2, vn, dis2,
                                 bn=(stats, pg.reshape(1, H), pb_.reshape(1, H)))
        raw = _sc_spmm(hs, row2d, col2d, zeros2d)
        pre, stats = _tc_post(raw, c2, W, hs, dis2, cb.reshape(1, H))
        if l < 2:
            vn = _tc_vnmlp(pooled, vn, *mlps[l], gmask2)

    return _tc_final(pre, stats, bn2_g.reshape(1, H), bn2_b.reshape(1, H))


# speedup vs baseline: 11.0874x; 1.2089x over previous
import functools

import jax
import jax.numpy as jnp
from jax import lax
from jax.experimental import pallas as pl
from jax.experimental.pallas import tpu as pltpu
from jax.experimental.pallas import tpu_sc as plsc

N = 10000
E = 320000
H = 128
NG = 64
NC = 2
NS = 16
NW = NC * NS
ER = E // 128
_G8 = 8
_NGRP = -(-ER // _G8)
_BASE = _NGRP // NW
_REMW = _NGRP - _BASE * NW
_GP1 = (_BASE + 1) * _G8
_GP0 = _BASE * _G8
RBMAX = _GP1
_PADROWS = _REMW * _GP1 + _GP0 * (NW - 1 - _REMW) + RBMAX

_MESH = plsc.VectorSubcoreMesh(core_axis_name="c", subcore_axis_name="s",
                               num_cores=NC, num_subcores=NS)


def _worker(c, s):
    w = s * NC + c
    start = jnp.where(w < _REMW, _GP1 * w, _REMW * _GP1 + _GP0 * (w - _REMW))
    cap = jnp.where(w < _REMW, _GP1, _GP0)
    nrows = jnp.minimum(cap, ER - start)
    return start, nrows


_RING = 2
_RING_DS = 4


def _sc_deg(row2d, ones128, zeros_n):
    @functools.partial(
        pl.kernel,
        out_type=jax.ShapeDtypeStruct((NC, N), jnp.float32),
        mesh=_MESH,
        scratch_types=[pltpu.VMEM((RBMAX, 128), jnp.int32),
                       pltpu.VMEM((128,), jnp.float32),
                       pltpu.VMEM_SHARED((N,), jnp.float32)],
    )
    def k(row_h, ones_h, zeros_h, out_h, idx_all, ones_v, acc_sh):
        c = lax.axis_index("c")
        s = lax.axis_index("s")
        @pl.when(s == 0)
        def _():
            pltpu.sync_copy(zeros_h, acc_sh)
        pltpu.sync_copy(ones_h, ones_v)
        start, nrows = _worker(c, s)
        pltpu.sync_copy(row_h.at[pl.ds(start, RBMAX)], idx_all)
        plsc.subcore_barrier()

        def body(i, carry):
            @pl.when(i < nrows)
            def _():
                pltpu.sync_copy(ones_v, acc_sh.at[idx_all.at[i]], add=True)
            return carry

        lax.fori_loop(0, RBMAX, body, 0)
        plsc.subcore_barrier()
        @pl.when(s == 0)
        def _():
            pltpu.sync_copy(acc_sh, out_h.at[c])

    return k(row2d, ones128, zeros_n)


def _sc_dsrow(row2d, dis):
    @functools.partial(
        pl.kernel,
        out_type=jax.ShapeDtypeStruct((E,), jnp.float32),
        mesh=_MESH,
        scratch_types=[pltpu.VMEM((RBMAX, 128), jnp.int32),
                       pltpu.VMEM((128,), jnp.float32)],
    )
    def k(row_h, dis_h, out_h, idx_all, ds_v):
        c = lax.axis_index("c")
        s = lax.axis_index("s")
        start, nrows = _worker(c, s)
        pltpu.sync_copy(row_h.at[pl.ds(start, RBMAX)], idx_all)

        def body(i, carry):
            @pl.when(i < nrows)
            def _():
                pltpu.sync_copy(dis_h.at[idx_all.at[i]], ds_v)
                pltpu.sync_copy(ds_v, out_h.at[pl.ds((start + i) * 128, 128)])
            return carry

        lax.fori_loop(0, RBMAX, body, 0)

    return k(row2d, dis)


def _sc_scatter_ea(ea3, col2d, zeros2d):
    @functools.partial(
        pl.kernel,
        out_type=jax.ShapeDtypeStruct((NC, N, H), jnp.float32),
        mesh=_MESH,
        scratch_types=[pltpu.VMEM((RBMAX, 128), jnp.int32),
                       pltpu.VMEM((128, H), jnp.float32),
                       pltpu.VMEM_SHARED((N, H), jnp.float32)],
    )
    def k(ea_h, col_h, z2_h, out_h, idx_all, rows_v, acc_sh):
        c = lax.axis_index("c")
        s = lax.axis_index("s")
        @pl.when(s == 0)
        def _():
            pltpu.sync_copy(z2_h, acc_sh)
        start, nrows = _worker(c, s)
        pltpu.sync_copy(col_h.at[pl.ds(start, RBMAX)], idx_all)
        plsc.subcore_barrier()

        def body(i, carry):
            @pl.when(i < nrows)
            def _():
                pltpu.sync_copy(ea_h.at[start + i], rows_v)
                pltpu.sync_copy(rows_v, acc_sh.at[idx_all.at[i]], add=True)
            return carry

        lax.fori_loop(0, RBMAX, body, 0)
        plsc.subcore_barrier()
        @pl.when(s == 0)
        def _():
            pltpu.sync_copy(acc_sh, out_h.at[c])

    return k(ea3, col2d, zeros2d)


def _sc_spmm(hs, row2d, col2d, zeros2d):
    @functools.partial(
        pl.kernel,
        out_type=jax.ShapeDtypeStruct((NC, N, H), jnp.float32),
        mesh=_MESH,
        scratch_types=[pltpu.VMEM((RBMAX, 128), jnp.int32),
                       pltpu.VMEM((RBMAX, 128), jnp.int32),
                       pltpu.VMEM((128, H), jnp.float32),
                       pltpu.VMEM_SHARED((N, H), jnp.float32)],
    )
    def k(hs_h, row_h, col_h, z2_h, out_h, idxr_all, idxc_all, rows_v, acc_sh):
        c = lax.axis_index("c")
        s = lax.axis_index("s")
        @pl.when(s == 0)
        def _():
            pltpu.sync_copy(z2_h, acc_sh)
        start, nrows = _worker(c, s)
        pltpu.sync_copy(row_h.at[pl.ds(start, RBMAX)], idxr_all)
        pltpu.sync_copy(col_h.at[pl.ds(start, RBMAX)], idxc_all)
        plsc.subcore_barrier()

        def body(i, carry):
            @pl.when(i < nrows)
            def _():
                pltpu.sync_copy(hs_h.at[idxr_all.at[i]], rows_v)
                pltpu.sync_copy(rows_v, acc_sh.at[idxc_all.at[i]], add=True)
            return carry

        lax.fori_loop(0, RBMAX, body, 0)
        plsc.subcore_barrier()
        @pl.when(s == 0)
        def _():
            pltpu.sync_copy(acc_sh, out_h.at[c])

    return k(hs, row2d, col2d, zeros2d)


_BLK = 400
_G = N // _BLK


def _tc_ea_proj(ea, pw, pb2, ds):
    blk = 2000

    def body(ea_ref, pw_ref, pb_ref, ds_ref, o_ref):
        p = jnp.dot(ea_ref[...], pw_ref[...],
                    preferred_element_type=jnp.float32) + pb_ref[...]
        o_ref[...] = p * ds_ref[...]

    return pl.pallas_call(
        body, grid=(E // blk,),
        in_specs=[pl.BlockSpec((blk, H), lambda i: (i, 0)),
                  pl.BlockSpec((H, H), lambda i: (0, 0)),
                  pl.BlockSpec((1, H), lambda i: (0, 0)),
                  pl.BlockSpec((blk, 1), lambda i: (i, 0))],
        out_specs=pl.BlockSpec((blk, H), lambda i: (i, 0)),
        out_shape=jax.ShapeDtypeStruct((E, H), jnp.float32),
    )(ea, pw, pb2, ds.reshape(E, 1))


def _tc_pre(src, batch2, vn, dis2, proj=None, bn=None):
    def body(src_ref, b_ref, vn_ref, dis_ref, e1_ref, e2_ref, e3_ref,
             hs_ref, pool_ref, acc):
        i = pl.program_id(0)
        if proj is not None:
            hbase = jnp.dot(src_ref[...], e1_ref[...],
                            preferred_element_type=jnp.float32) + e2_ref[...]
        else:
            st = e1_ref[...]
            mu = st[0:1] * (1.0 / N)
            var = st[1:2] * (1.0 / N) - mu * mu
            hbase = (src_ref[...] - mu) * lax.rsqrt(var + 1e-5) * e2_ref[...] + e3_ref[...]
            hbase = jnp.maximum(hbase, 0.0)
        ids = lax.broadcasted_iota(jnp.int32, (_BLK, NG), 1)
        oh = (b_ref[...] == ids).astype(jnp.float32)
        hl = hbase + jnp.dot(oh, vn_ref[...], preferred_element_type=jnp.float32,
                             precision=lax.Precision.HIGHEST)
        hs_ref[...] = dis_ref[...] * hl
        p = lax.dot_general(oh, hl, (((0,), (0,)), ((), ())),
                            preferred_element_type=jnp.float32,
                            precision=lax.Precision.HIGHEST)
        @pl.when(i == 0)
        def _():
            acc[...] = jnp.zeros_like(acc)
        acc[...] += p
        pool_ref[...] = acc[...]

    if proj is not None:
        e_specs = [pl.BlockSpec((H, H), lambda i: (0, 0)),
                   pl.BlockSpec((1, H), lambda i: (0, 0)),
                   pl.BlockSpec((1, H), lambda i: (0, 0))]
        extra = [proj[0], proj[1], proj[1]]
    else:
        e_specs = [pl.BlockSpec((2, H), lambda i: (0, 0)),
                   pl.BlockSpec((1, H), lambda i: (0, 0)),
                   pl.BlockSpec((1, H), lambda i: (0, 0))]
        extra = [bn[0], bn[1], bn[2]]

    return pl.pallas_call(
        body, grid=(_G,),
        in_specs=[pl.BlockSpec((_BLK, H), lambda i: (i, 0)),
                  pl.BlockSpec((_BLK, 1), lambda i: (i, 0)),
                  pl.BlockSpec((NG, H), lambda i: (0, 0)),
                  pl.BlockSpec((_BLK, 1), lambda i: (i, 0))] + e_specs,
        out_specs=[pl.BlockSpec((_BLK, H), lambda i: (i, 0)),
                   pl.BlockSpec((NG, H), lambda i: (0, 0))],
        out_shape=[jax.ShapeDtypeStruct((N, H), jnp.float32),
                   jax.ShapeDtypeStruct((NG, H), jnp.float32)],
        scratch_shapes=[pltpu.VMEM((NG, H), jnp.float32)],
    )(src, batch2, vn, dis2, *extra)


def _tc_post(raw, c2, W, hs, dis2, cb2):
    def body(raw_ref, c_ref, W_ref, hs_ref, dis_ref, cb_ref,
             pre_ref, st_ref, acc):
        i = pl.program_id(0)
        r = raw_ref[0] + raw_ref[1] + c_ref[0] + c_ref[1]
        aggr = dis_ref[...] * r + dis_ref[...] * hs_ref[...]
        pre = jnp.dot(aggr, W_ref[...],
                      preferred_element_type=jnp.float32) + cb_ref[...]
        pre_ref[...] = pre
        @pl.when(i == 0)
        def _():
            acc[...] = jnp.zeros_like(acc)
        acc[0:1] += pre.sum(0, keepdims=True)
        acc[1:2] += (pre * pre).sum(0, keepdims=True)
        st_ref[...] = acc[...]

    return pl.pallas_call(
        body, grid=(_G,),
        in_specs=[pl.BlockSpec((NC, _BLK, H), lambda i: (0, i, 0)),
                  pl.BlockSpec((NC, _BLK, H), lambda i: (0, i, 0)),
                  pl.BlockSpec((H, H), lambda i: (0, 0)),
                  pl.BlockSpec((_BLK, H), lambda i: (i, 0)),
                  pl.BlockSpec((_BLK, 1), lambda i: (i, 0)),
                  pl.BlockSpec((1, H), lambda i: (0, 0))],
        out_specs=[pl.BlockSpec((_BLK, H), lambda i: (i, 0)),
                   pl.BlockSpec((2, H), lambda i: (0, 0))],
        out_shape=[jax.ShapeDtypeStruct((N, H), jnp.float32),
                   jax.ShapeDtypeStruct((2, H), jnp.float32)],
        scratch_shapes=[pltpu.VMEM((2, H), jnp.float32)],
    )(raw, c2, W, hs, dis2, cb2)


def _tc_vnmlp(pooled, vn, w1, b1, g1, bb1, w2, b2, g2, bb2, gmask2):
    def body(p_ref, vn_ref, w1_ref, b1_ref, g1_ref, bb1_ref,
             w2_ref, b2_ref, g2_ref, bb2_ref, gm_ref, o_ref):
        gm = gm_ref[...]
        cnt = gm.sum()
        t = p_ref[...] + vn_ref[...]
        t = jnp.dot(t, w1_ref[...], preferred_element_type=jnp.float32) + b1_ref[...]
        mu = (t * gm).sum(0, keepdims=True) / cnt
        var = (((t - mu) ** 2) * gm).sum(0, keepdims=True) / cnt
        t = jnp.maximum((t - mu) * lax.rsqrt(var + 1e-5) * g1_ref[...] + bb1_ref[...], 0.0)
        t = jnp.dot(t, w2_ref[...], preferred_element_type=jnp.float32) + b2_ref[...]
        mu2 = (t * gm).sum(0, keepdims=True) / cnt
        var2 = (((t - mu2) ** 2) * gm).sum(0, keepdims=True) / cnt
        o_ref[...] = jnp.maximum((t - mu2) * lax.rsqrt(var2 + 1e-5) * g2_ref[...] + bb2_ref[...], 0.0)

    return pl.pallas_call(
        body, out_shape=jax.ShapeDtypeStruct((NG, H), jnp.float32),
    )(pooled, vn, w1, b1.reshape(1, -1), g1.reshape(1, -1), bb1.reshape(1, -1),
      w2, b2.reshape(1, -1), g2.reshape(1, -1), bb2.reshape(1, -1), gmask2)


def _tc_final(pre, stats, g2, b2):
    def body(pre_ref, st_ref, g_ref, b_ref, o_ref):
        st = st_ref[...]
        mu = st[0:1] * (1.0 / N)
        var = st[1:2] * (1.0 / N) - mu * mu
        o_ref[...] = (pre_ref[...] - mu) * lax.rsqrt(var + 1e-5) * g_ref[...] + b_ref[...]

    return pl.pallas_call(
        body, grid=(_G,),
        in_specs=[pl.BlockSpec((_BLK, H), lambda i: (i, 0)),
                  pl.BlockSpec((2, H), lambda i: (0, 0)),
                  pl.BlockSpec((1, H), lambda i: (0, 0)),
                  pl.BlockSpec((1, H), lambda i: (0, 0))],
        out_specs=pl.BlockSpec((_BLK, H), lambda i: (i, 0)),
        out_shape=jax.ShapeDtypeStruct((N, H), jnp.float32),
    )(pre, stats, g2, b2)


def kernel(x, edge_index, edge_attr, batch,
           proj_w, proj_b, vn_emb,
           conv0_w, conv0_b, bn0_g, bn0_b,
           conv1_w, conv1_b, bn1_g, bn1_b,
           conv2_w, conv2_b, bn2_g, bn2_b,
           mlp0_w1, mlp0_b1, mlp0_bn1_g, mlp0_bn1_b, mlp0_w2, mlp0_b2, mlp0_bn2_g, mlp0_bn2_b,
           mlp1_w1, mlp1_b1, mlp1_bn1_g, mlp1_bn1_b, mlp1_w2, mlp1_b2, mlp1_bn2_g, mlp1_bn2_b):
    row2d = edge_index[0].astype(jnp.int32).reshape(ER, 128)
    col2d = edge_index[1].astype(jnp.int32).reshape(ER, 128)
    row2d = jnp.pad(row2d, ((0, _PADROWS - ER), (0, 0)))
    col2d = jnp.pad(col2d, ((0, _PADROWS - ER), (0, 0)))
    zeros_n = jnp.zeros((N,), jnp.float32)
    zeros2d = jnp.zeros((N, H), jnp.float32)
    ones128 = jnp.ones((128,), jnp.float32)

    deg2 = _sc_deg(row2d, ones128, zeros_n)
    dis = (deg2[0] + deg2[1] + 1.0) ** -0.5
    dis2 = dis.reshape(N, 1)
    dsrow = _sc_dsrow(row2d, dis)
    pb2 = proj_b.reshape(1, H)
    ea_s = _tc_ea_proj(edge_attr, proj_w, pb2, dsrow)
    c2 = _sc_scatter_ea(ea_s.reshape(ER, 128, H), col2d, zeros2d)

    batch2 = batch.astype(jnp.int32).reshape(N, 1)
    gmask2 = (jnp.arange(NG) < batch[-1] + 1).astype(jnp.float32).reshape(NG, 1)
    vn = jnp.broadcast_to(vn_emb, (NG, H))

    convs = [(conv0_w, conv0_b, bn0_g, bn0_b),
             (conv1_w, conv1_b, bn1_g, bn1_b),
             (conv2_w, conv2_b, bn2_g, bn2_b)]
    mlps = [(mlp0_w1, mlp0_b1, mlp0_bn1_g, mlp0_bn1_b, mlp0_w2, mlp0_b2, mlp0_bn2_g, mlp0_bn2_b),
            (mlp1_w1, mlp1_b1, mlp1_bn1_g, mlp1_bn1_b, mlp1_w2, mlp1_b2, mlp1_bn2_g, mlp1_bn2_b)]

    pre = None
    stats = None
    for l in range(3):
        W, cb, _, _ = convs[l]
        if l == 0:
            hs, pooled = _tc_pre(x, batch2, vn, dis2, proj=(proj_w, pb2))
        else:
            pg, pb_ = convs[l - 1][2], convs[l - 1][3]
            hs, pooled = _tc_pre(pre, batch2, vn, dis2,
                                 bn=(stats, pg.reshape(1, H), pb_.reshape(1, H)))
        raw = _sc_spmm(hs, row2d, col2d, zeros2d)
        pre, stats = _tc_post(raw, c2, W, hs, dis2, cb.reshape(1, H))
        if l < 2:
            vn = _tc_vnmlp(pooled, vn, *mlps[l], gmask2)

    return _tc_final(pre, stats, bn2_g.reshape(1, H), bn2_b.reshape(1, H))


# speedup vs baseline: 11.8574x; 1.0694x over previous
import functools

import jax
import jax.numpy as jnp
from jax import lax
from jax.experimental import pallas as pl
from jax.experimental.pallas import tpu as pltpu
from jax.experimental.pallas import tpu_sc as plsc

N = 10000
E = 320000
H = 128
NG = 64
NC = 2
NS = 16
NW = NC * NS
ER = E // 128
_G8 = 8
_NGRP = -(-ER // _G8)
_BASE = _NGRP // NW
_REMW = _NGRP - _BASE * NW
_GP1 = (_BASE + 1) * _G8
_GP0 = _BASE * _G8
RBMAX = _GP1
_PADROWS = _REMW * _GP1 + _GP0 * (NW - 1 - _REMW) + RBMAX

_MESH = plsc.VectorSubcoreMesh(core_axis_name="c", subcore_axis_name="s",
                               num_cores=NC, num_subcores=NS)


def _worker(c, s):
    w = s * NC + c
    start = jnp.where(w < _REMW, _GP1 * w, _REMW * _GP1 + _GP0 * (w - _REMW))
    cap = jnp.where(w < _REMW, _GP1, _GP0)
    nrows = jnp.minimum(cap, ER - start)
    return start, nrows


_RING = 2
_RING_DS = 4


def _sc_deg(row2d, ones128, zeros_n):
    @functools.partial(
        pl.kernel,
        out_type=jax.ShapeDtypeStruct((NC, N), jnp.float32),
        mesh=_MESH,
        scratch_types=[pltpu.VMEM((RBMAX, 128), jnp.int32),
                       pltpu.VMEM((128,), jnp.float32),
                       pltpu.VMEM_SHARED((N,), jnp.float32)],
    )
    def k(row_h, ones_h, zeros_h, out_h, idx_all, ones_v, acc_sh):
        c = lax.axis_index("c")
        s = lax.axis_index("s")
        @pl.when(s == 0)
        def _():
            pltpu.sync_copy(zeros_h, acc_sh)
        pltpu.sync_copy(ones_h, ones_v)
        start, nrows = _worker(c, s)
        pltpu.sync_copy(row_h.at[pl.ds(start, RBMAX)], idx_all)
        plsc.subcore_barrier()

        def body(i, carry):
            @pl.when(i < nrows)
            def _():
                pltpu.sync_copy(ones_v, acc_sh.at[idx_all.at[i]], add=True)
            return carry

        lax.fori_loop(0, RBMAX, body, 0)
        plsc.subcore_barrier()
        @pl.when(s == 0)
        def _():
            pltpu.sync_copy(acc_sh, out_h.at[c])

    return k(row2d, ones128, zeros_n)


def _sc_dsrow(row2d, dis):
    @functools.partial(
        pl.kernel,
        out_type=jax.ShapeDtypeStruct((E,), jnp.float32),
        mesh=_MESH,
        scratch_types=[pltpu.VMEM((RBMAX, 128), jnp.int32),
                       pltpu.VMEM((128,), jnp.float32)],
    )
    def k(row_h, dis_h, out_h, idx_all, ds_v):
        c = lax.axis_index("c")
        s = lax.axis_index("s")
        start, nrows = _worker(c, s)
        pltpu.sync_copy(row_h.at[pl.ds(start, RBMAX)], idx_all)

        def body(i, carry):
            @pl.when(i < nrows)
            def _():
                pltpu.sync_copy(dis_h.at[idx_all.at[i]], ds_v)
                pltpu.sync_copy(ds_v, out_h.at[pl.ds((start + i) * 128, 128)])
            return carry

        lax.fori_loop(0, RBMAX, body, 0)

    return k(row2d, dis)


def _sc_scatter_ea(ea3, col2d, zeros2d):
    @functools.partial(
        pl.kernel,
        out_type=jax.ShapeDtypeStruct((NC, N, H), jnp.float32),
        mesh=_MESH,
        scratch_types=[pltpu.VMEM((RBMAX, 128), jnp.int32),
                       pltpu.VMEM((2, 128, H), jnp.float32),
                       pltpu.VMEM_SHARED((N, H), jnp.float32),
                       pltpu.SemaphoreType.DMA,
                       pltpu.SemaphoreType.DMA],
    )
    def k(ea_h, col_h, z2_h, out_h, idx_all, rows_v, acc_sh, sem0, sem1):
        c = lax.axis_index("c")
        s = lax.axis_index("s")
        @pl.when(s == 0)
        def _():
            pltpu.sync_copy(z2_h, acc_sh)
        start, nrows = _worker(c, s)
        pltpu.sync_copy(col_h.at[pl.ds(start, RBMAX)], idx_all)
        plsc.subcore_barrier()

        def body(i, carry):
            j0 = 2 * i
            j1 = 2 * i + 1
            d0 = pltpu.make_async_copy(ea_h.at[start + j0], rows_v.at[0], sem0)
            d1 = pltpu.make_async_copy(ea_h.at[start + j1], rows_v.at[1], sem1)
            @pl.when(j0 < nrows)
            def _():
                d0.start()
            @pl.when(j1 < nrows)
            def _():
                d1.start()
            @pl.when(j0 < nrows)
            def _():
                d0.wait()
                pltpu.sync_copy(rows_v.at[0], acc_sh.at[idx_all.at[j0]],
                                add=True)
            @pl.when(j1 < nrows)
            def _():
                d1.wait()
                pltpu.sync_copy(rows_v.at[1], acc_sh.at[idx_all.at[j1]],
                                add=True)
            return carry

        lax.fori_loop(0, RBMAX // 2, body, 0)
        plsc.subcore_barrier()
        @pl.when(s == 0)
        def _():
            pltpu.sync_copy(acc_sh, out_h.at[c])

    return k(ea3, col2d, zeros2d)


def _sc_spmm(hs, row2d, col2d, zeros2d):
    HALF = RBMAX // 2

    @functools.partial(
        pl.kernel,
        out_type=jax.ShapeDtypeStruct((NC, N, H), jnp.float32),
        mesh=_MESH,
        scratch_types=[pltpu.VMEM((HALF, 128), jnp.int32),
                       pltpu.VMEM((HALF, 128), jnp.int32),
                       pltpu.VMEM((2, 128, H), jnp.float32),
                       pltpu.VMEM_SHARED((N, H), jnp.float32),
                       pltpu.SemaphoreType.DMA,
                       pltpu.SemaphoreType.DMA],
    )
    def k(hs_h, row_h, col_h, z2_h, out_h, idxr_v, idxc_v, rows_v,
          acc_sh, sem0, sem1):
        c = lax.axis_index("c")
        s = lax.axis_index("s")
        @pl.when(s == 0)
        def _():
            pltpu.sync_copy(z2_h, acc_sh)
        start, nrows = _worker(c, s)
        plsc.subcore_barrier()

        for p in range(2):
            base = p * HALF
            pltpu.sync_copy(row_h.at[pl.ds(start + base, HALF)], idxr_v)
            pltpu.sync_copy(col_h.at[pl.ds(start + base, HALF)], idxc_v)
            nloc = jnp.clip(nrows - base, 0, HALF)

            def body(i, carry):
                j0 = 2 * i
                j1 = 2 * i + 1
                d0 = pltpu.make_async_copy(hs_h.at[idxr_v.at[j0]],
                                           rows_v.at[0], sem0)
                d1 = pltpu.make_async_copy(hs_h.at[idxr_v.at[j1]],
                                           rows_v.at[1], sem1)
                @pl.when(j0 < nloc)
                def _():
                    d0.start()
                @pl.when(j1 < nloc)
                def _():
                    d1.start()
                @pl.when(j0 < nloc)
                def _():
                    d0.wait()
                    pltpu.sync_copy(rows_v.at[0], acc_sh.at[idxc_v.at[j0]],
                                    add=True)
                @pl.when(j1 < nloc)
                def _():
                    d1.wait()
                    pltpu.sync_copy(rows_v.at[1], acc_sh.at[idxc_v.at[j1]],
                                    add=True)
                return carry

            lax.fori_loop(0, HALF // 2, body, 0)

        plsc.subcore_barrier()
        @pl.when(s == 0)
        def _():
            pltpu.sync_copy(acc_sh, out_h.at[c])

    return k(hs, row2d, col2d, zeros2d)


_BLK = 400
_G = N // _BLK


def _tc_ea_proj(ea, pw, pb2, ds):
    blk = 2000

    def body(ea_ref, pw_ref, pb_ref, ds_ref, o_ref):
        p = jnp.dot(ea_ref[...], pw_ref[...],
                    preferred_element_type=jnp.float32) + pb_ref[...]
        o_ref[...] = p * ds_ref[...]

    return pl.pallas_call(
        body, grid=(E // blk,),
        in_specs=[pl.BlockSpec((blk, H), lambda i: (i, 0)),
                  pl.BlockSpec((H, H), lambda i: (0, 0)),
                  pl.BlockSpec((1, H), lambda i: (0, 0)),
                  pl.BlockSpec((blk, 1), lambda i: (i, 0))],
        out_specs=pl.BlockSpec((blk, H), lambda i: (i, 0)),
        out_shape=jax.ShapeDtypeStruct((E, H), jnp.float32),
    )(ea, pw, pb2, ds.reshape(E, 1))


def _tc_pre(src, batch2, vn, dis2, proj=None, bn=None):
    def body(src_ref, b_ref, vn_ref, dis_ref, e1_ref, e2_ref, e3_ref,
             hs_ref, pool_ref, acc):
        i = pl.program_id(0)
        if proj is not None:
            hbase = jnp.dot(src_ref[...], e1_ref[...],
                            preferred_element_type=jnp.float32) + e2_ref[...]
        else:
            st = e1_ref[...]
            mu = st[0:1] * (1.0 / N)
            var = st[1:2] * (1.0 / N) - mu * mu
            hbase = (src_ref[...] - mu) * lax.rsqrt(var + 1e-5) * e2_ref[...] + e3_ref[...]
            hbase = jnp.maximum(hbase, 0.0)
        ids = lax.broadcasted_iota(jnp.int32, (_BLK, NG), 1)
        oh = (b_ref[...] == ids).astype(jnp.float32)
        hl = hbase + jnp.dot(oh, vn_ref[...], preferred_element_type=jnp.float32,
                             precision=lax.Precision.HIGHEST)
        hs_ref[...] = dis_ref[...] * hl
        p = lax.dot_general(oh, hl, (((0,), (0,)), ((), ())),
                            preferred_element_type=jnp.float32,
                            precision=lax.Precision.HIGHEST)
        @pl.when(i == 0)
        def _():
            acc[...] = jnp.zeros_like(acc)
        acc[...] += p
        pool_ref[...] = acc[...]

    if proj is not None:
        e_specs = [pl.BlockSpec((H, H), lambda i: (0, 0)),
                   pl.BlockSpec((1, H), lambda i: (0, 0)),
                   pl.BlockSpec((1, H), lambda i: (0, 0))]
        extra = [proj[0], proj[1], proj[1]]
    else:
        e_specs = [pl.BlockSpec((2, H), lambda i: (0, 0)),
                   pl.BlockSpec((1, H), lambda i: (0, 0)),
                   pl.BlockSpec((1, H), lambda i: (0, 0))]
        extra = [bn[0], bn[1], bn[2]]

    return pl.pallas_call(
        body, grid=(_G,),
        in_specs=[pl.BlockSpec((_BLK, H), lambda i: (i, 0)),
                  pl.BlockSpec((_BLK, 1), lambda i: (i, 0)),
                  pl.BlockSpec((NG, H), lambda i: (0, 0)),
                  pl.BlockSpec((_BLK, 1), lambda i: (i, 0))] + e_specs,
        out_specs=[pl.BlockSpec((_BLK, H), lambda i: (i, 0)),
                   pl.BlockSpec((NG, H), lambda i: (0, 0))],
        out_shape=[jax.ShapeDtypeStruct((N, H), jnp.float32),
                   jax.ShapeDtypeStruct((NG, H), jnp.float32)],
        scratch_shapes=[pltpu.VMEM((NG, H), jnp.float32)],
    )(src, batch2, vn, dis2, *extra)


def _tc_post(raw, c2, W, hs, dis2, cb2):
    def body(raw_ref, c_ref, W_ref, hs_ref, dis_ref, cb_ref,
             pre_ref, st_ref, acc):
        i = pl.program_id(0)
        r = raw_ref[0] + raw_ref[1] + c_ref[0] + c_ref[1]
        aggr = dis_ref[...] * r + dis_ref[...] * hs_ref[...]
        pre = jnp.dot(aggr, W_ref[...],
                      preferred_element_type=jnp.float32) + cb_ref[...]
        pre_ref[...] = pre
        @pl.when(i == 0)
        def _():
            acc[...] = jnp.zeros_like(acc)
        acc[0:1] += pre.sum(0, keepdims=True)
        acc[1:2] += (pre * pre).sum(0, keepdims=True)
        st_ref[...] = acc[...]

    return pl.pallas_call(
        body, grid=(_G,),
        in_specs=[pl.BlockSpec((NC, _BLK, H), lambda i: (0, i, 0)),
                  pl.BlockSpec((NC, _BLK, H), lambda i: (0, i, 0)),
                  pl.BlockSpec((H, H), lambda i: (0, 0)),
                  pl.BlockSpec((_BLK, H), lambda i: (i, 0)),
                  pl.BlockSpec((_BLK, 1), lambda i: (i, 0)),
                  pl.BlockSpec((1, H), lambda i: (0, 0))],
        out_specs=[pl.BlockSpec((_BLK, H), lambda i: (i, 0)),
                   pl.BlockSpec((2, H), lambda i: (0, 0))],
        out_shape=[jax.ShapeDtypeStruct((N, H), jnp.float32),
                   jax.ShapeDtypeStruct((2, H), jnp.float32)],
        scratch_shapes=[pltpu.VMEM((2, H), jnp.float32)],
    )(raw, c2, W, hs, dis2, cb2)


def _tc_vnmlp(pooled, vn, w1, b1, g1, bb1, w2, b2, g2, bb2, gmask2):
    def body(p_ref, vn_ref, w1_ref, b1_ref, g1_ref, bb1_ref,
             w2_ref, b2_ref, g2_ref, bb2_ref, gm_ref, o_ref):
        gm = gm_ref[...]
        cnt = gm.sum()
        t = p_ref[...] + vn_ref[...]
        t = jnp.dot(t, w1_ref[...], preferred_element_type=jnp.float32) + b1_ref[...]
        mu = (t * gm).sum(0, keepdims=True) / cnt
        var = (((t - mu) ** 2) * gm).sum(0, keepdims=True) / cnt
        t = jnp.maximum((t - mu) * lax.rsqrt(var + 1e-5) * g1_ref[...] + bb1_ref[...], 0.0)
        t = jnp.dot(t, w2_ref[...], preferred_element_type=jnp.float32) + b2_ref[...]
        mu2 = (t * gm).sum(0, keepdims=True) / cnt
        var2 = (((t - mu2) ** 2) * gm).sum(0, keepdims=True) / cnt
        o_ref[...] = jnp.maximum((t - mu2) * lax.rsqrt(var2 + 1e-5) * g2_ref[...] + bb2_ref[...], 0.0)

    return pl.pallas_call(
        body, out_shape=jax.ShapeDtypeStruct((NG, H), jnp.float32),
    )(pooled, vn, w1, b1.reshape(1, -1), g1.reshape(1, -1), bb1.reshape(1, -1),
      w2, b2.reshape(1, -1), g2.reshape(1, -1), bb2.reshape(1, -1), gmask2)


def _tc_final(pre, stats, g2, b2):
    def body(pre_ref, st_ref, g_ref, b_ref, o_ref):
        st = st_ref[...]
        mu = st[0:1] * (1.0 / N)
        var = st[1:2] * (1.0 / N) - mu * mu
        o_ref[...] = (pre_ref[...] - mu) * lax.rsqrt(var + 1e-5) * g_ref[...] + b_ref[...]

    return pl.pallas_call(
        body, grid=(_G,),
        in_specs=[pl.BlockSpec((_BLK, H), lambda i: (i, 0)),
                  pl.BlockSpec((2, H), lambda i: (0, 0)),
                  pl.BlockSpec((1, H), lambda i: (0, 0)),
                  pl.BlockSpec((1, H), lambda i: (0, 0))],
        out_specs=pl.BlockSpec((_BLK, H), lambda i: (i, 0)),
        out_shape=jax.ShapeDtypeStruct((N, H), jnp.float32),
    )(pre, stats, g2, b2)


def kernel(x, edge_index, edge_attr, batch,
           proj_w, proj_b, vn_emb,
           conv0_w, conv0_b, bn0_g, bn0_b,
           conv1_w, conv1_b, bn1_g, bn1_b,
           conv2_w, conv2_b, bn2_g, bn2_b,
           mlp0_w1, mlp0_b1, mlp0_bn1_g, mlp0_bn1_b, mlp0_w2, mlp0_b2, mlp0_bn2_g, mlp0_bn2_b,
           mlp1_w1, mlp1_b1, mlp1_bn1_g, mlp1_bn1_b, mlp1_w2, mlp1_b2, mlp1_bn2_g, mlp1_bn2_b):
    row2d = edge_index[0].astype(jnp.int32).reshape(ER, 128)
    col2d = edge_index[1].astype(jnp.int32).reshape(ER, 128)
    row2d = jnp.pad(row2d, ((0, _PADROWS - ER), (0, 0)))
    col2d = jnp.pad(col2d, ((0, _PADROWS - ER), (0, 0)))
    zeros_n = jnp.zeros((N,), jnp.float32)
    zeros2d = jnp.zeros((N, H), jnp.float32)
    ones128 = jnp.ones((128,), jnp.float32)

    deg2 = _sc_deg(row2d, ones128, zeros_n)
    dis = (deg2[0] + deg2[1] + 1.0) ** -0.5
    dis2 = dis.reshape(N, 1)
    dsrow = _sc_dsrow(row2d, dis)
    pb2 = proj_b.reshape(1, H)
    ea_s = _tc_ea_proj(edge_attr, proj_w, pb2, dsrow)
    c2 = _sc_scatter_ea(ea_s.reshape(ER, 128, H), col2d, zeros2d)

    batch2 = batch.astype(jnp.int32).reshape(N, 1)
    gmask2 = (jnp.arange(NG) < batch[-1] + 1).astype(jnp.float32).reshape(NG, 1)
    vn = jnp.broadcast_to(vn_emb, (NG, H))

    convs = [(conv0_w, conv0_b, bn0_g, bn0_b),
             (conv1_w, conv1_b, bn1_g, bn1_b),
             (conv2_w, conv2_b, bn2_g, bn2_b)]
    mlps = [(mlp0_w1, mlp0_b1, mlp0_bn1_g, mlp0_bn1_b, mlp0_w2, mlp0_b2, mlp0_bn2_g, mlp0_bn2_b),
            (mlp1_w1, mlp1_b1, mlp1_bn1_g, mlp1_bn1_b, mlp1_w2, mlp1_b2, mlp1_bn2_g, mlp1_bn2_b)]

    pre = None
    stats = None
    for l in range(3):
        W, cb, _, _ = convs[l]
        if l == 0:
            hs, pooled = _tc_pre(x, batch2, vn, dis2, proj=(proj_w, pb2))
        else:
            pg, pb_ = convs[l - 1][2], convs[l - 1][3]
            hs, pooled = _tc_pre(pre, batch2, vn, dis2,
                                 bn=(stats, pg.reshape(1, H), pb_.reshape(1, H)))
        raw = _sc_spmm(hs, row2d, col2d, zeros2d)
        pre, stats = _tc_post(raw, c2, W, hs, dis2, cb.reshape(1, H))
        if l < 2:
            vn = _tc_vnmlp(pooled, vn, *mlps[l], gmask2)

    return _tc_final(pre, stats, bn2_g.reshape(1, H), bn2_b.reshape(1, H))
